# baseline (device time: 61787 ns/iter reference)
import functools

import jax
import jax.numpy as jnp
from jax import lax
from jax.experimental import pallas as pl
from jax.experimental.pallas import tpu as pltpu

N_DEV = 8
SQ = 1024
SKV_SH = 1024
HQ = 8
DH = 128
D = HQ * DH
PW = D + 128
BLK = SQ // N_DEV
KT = 128
NKT = SKV_SH // KT
SCALE = 0.08838834764831843
BAND = 128
NGLOB = 32

_MESH = pl.DeviceIdType.MESH
_BF16 = jnp.bfloat16
_F32 = jnp.float32


def _block_partial_dev0(q, k_ref, v_ref, t):
    if t == 0:
        lo, n = 0, SKV_SH
    elif t == 1:
        lo, n = 0, 3 * KT
    else:
        lo = (t - 1) * KT
        n = min(t + 2, NKT) * KT - lo
    glob_extra = t >= 2
    qi = t * BLK + lax.broadcasted_iota(jnp.int32, (BLK, n), 0)
    kj = lo + lax.broadcasted_iota(jnp.int32, (BLK, n), 1)
    mask = (jnp.abs(qi - kj) <= BAND) | (kj < NGLOB) | (qi < NGLOB)
    cols = []
    dens = []
    for h in range(HQ):
        qh = q[t * BLK:(t + 1) * BLK, h * DH:(h + 1) * DH]
        kb = k_ref[lo:lo + n, h, :]
        s = lax.dot_general(
            qh, kb, (((1,), (1,)), ((), ())),
            preferred_element_type=_F32,
        ) * SCALE
        w = jnp.where(mask, jnp.exp(s), 0.0)
        num_h = jnp.dot(w, v_ref[lo:lo + n, h, :],
                        preferred_element_type=_F32)
        den_h = jnp.sum(w, axis=1, keepdims=True)
        if glob_extra:
            kg = k_ref[0:NGLOB, h, :]
            sg = lax.dot_general(
                qh, kg, (((1,), (1,)), ((), ())),
                preferred_element_type=_F32,
            ) * SCALE
            wg = jnp.exp(sg)
            num_h = num_h + jnp.dot(wg, v_ref[0:NGLOB, h, :],
                                    preferred_element_type=_F32)
            den_h = den_h + jnp.sum(wg, axis=1, keepdims=True)
        cols.append(num_h)
        dens.append(den_h)
    den = jnp.concatenate(dens, axis=1)
    pad = jnp.zeros((BLK, PW - D - HQ), _F32)
    return jnp.concatenate(cols + [den, pad], axis=1).astype(_BF16)


def _combine_finalize(my, wo_ref, o_ref, bsum_ref, agsrc_ref, agdst_ref,
                      s1_sems, ag_send_sems, ag_recv_sems):
    parts = []
    for h in range(HQ):
        n = bsum_ref[:, h * DH:(h + 1) * DH]
        d = bsum_ref[:, D + h:D + h + 1]
        parts.append(n / d)
    ctx = jnp.concatenate(parts, axis=1)
    oblk = jnp.dot(ctx, wo_ref[...], preferred_element_type=_F32)

    o_ref[0, pl.ds(my * BLK, BLK)] = oblk
    agsrc_ref[...] = oblk.astype(_BF16)

    for t in range(N_DEV):
        @pl.when(my != t)
        def _(t=t):
            pltpu.make_async_remote_copy(
                src_ref=agsrc_ref,
                dst_ref=agdst_ref.at[my],
                send_sem=ag_send_sems.at[t],
                recv_sem=ag_recv_sems.at[my],
                device_id=(t,), device_id_type=_MESH,
            ).start()

    for k in range(N_DEV):
        @pl.when(my != k)
        def _(k=k):
            pltpu.make_async_remote_copy(
                src_ref=agsrc_ref,
                dst_ref=agdst_ref.at[k],
                send_sem=s1_sems.at[0],
                recv_sem=ag_recv_sems.at[k],
                device_id=(0,), device_id_type=_MESH,
            ).wait_recv()
            o_ref[0, k * BLK:(k + 1) * BLK] = agdst_ref[k].astype(_F32)

    for t in range(N_DEV):
        @pl.when(my != t)
        def _(t=t):
            pltpu.make_async_remote_copy(
                src_ref=agsrc_ref,
                dst_ref=agdst_ref.at[0],
                send_sem=ag_send_sems.at[t],
                recv_sem=ag_recv_sems.at[0],
                device_id=(t,), device_id_type=_MESH,
            ).wait_send()


def _fused_body(x_ref, wq_ref, k_ref, v_ref, wo_ref, o_ref,
                pstage_ref, p32_ref, px_ref,
                c0_ref, cx_ref, bsum_ref, agsrc_ref, agdst_ref,
                c0_sems, cx_sems, s0_sems, s1_sems,
                ag_send_sems, ag_recv_sems):
    my = lax.axis_index("i")

    barrier_sem = pltpu.get_barrier_semaphore()
    for p in range(N_DEV):
        @pl.when(my != p)
        def _(p=p):
            pl.semaphore_signal(
                barrier_sem, inc=1, device_id=(p,), device_id_type=_MESH)
    pl.semaphore_wait(barrier_sem, N_DEV - 1)

    @pl.when(my == 0)
    def _():
        q = jnp.dot(x_ref[0], wq_ref[...], preferred_element_type=_F32)
        for t in range(1, N_DEV):
            pstage_ref[t - 1] = _block_partial_dev0(q, k_ref, v_ref, t)
            pltpu.make_async_remote_copy(
                src_ref=pstage_ref.at[t - 1],
                dst_ref=cx_ref.at[0],
                send_sem=s0_sems.at[t - 1],
                recv_sem=cx_sems.at[0],
                device_id=(t,), device_id_type=_MESH,
            ).start()
        bsum_ref[...] = _block_partial_dev0(q, k_ref, v_ref, 0).astype(_F32)

    @pl.when(my != 0)
    def _():
        qg = jnp.dot(x_ref[0, pl.ds(0, NGLOB)], wq_ref[...],
                     preferred_element_type=_F32)
        cols = []
        dens = []
        for h in range(HQ):
            qh = qg[:, h * DH:(h + 1) * DH]
            kh = k_ref[:, h, :]
            s = lax.dot_general(
                qh, kh, (((1,), (1,)), ((), ())),
                preferred_element_type=_F32,
            ) * SCALE
            w = jnp.exp(s)
            cols.append(jnp.dot(w, v_ref[:, h, :],
                                preferred_element_type=_F32))
            dens.append(jnp.sum(w, axis=1, keepdims=True))
        den = jnp.concatenate(dens, axis=1)
        pad = jnp.zeros((NGLOB, PW - D - HQ), _F32)
        p32_ref[...] = jnp.concatenate(cols + [den, pad], axis=1).astype(_BF16)

    for p in range(1, N_DEV):
        @pl.when(my == p)
        def _(p=p):
            pltpu.make_async_remote_copy(
                src_ref=p32_ref,
                dst_ref=c0_ref.at[p - 1],
                send_sem=s1_sems.at[0],
                recv_sem=c0_sems.at[p - 1],
                device_id=(0,), device_id_type=_MESH,
            ).start()

    @pl.when(my == 1)
    def _():
        q9 = jnp.dot(x_ref[0, pl.ds(7 * BLK, BLK)], wq_ref[...],
                     preferred_element_type=_F32)
        qi = 7 * BLK + lax.broadcasted_iota(jnp.int32, (BLK, KT), 0)
        kj = SKV_SH + lax.broadcasted_iota(jnp.int32, (BLK, KT), 1)
        mask = jnp.abs(qi - kj) <= BAND
        cols = []
        dens = []
        for h in range(HQ):
            qh = q9[:, h * DH:(h + 1) * DH]
            kh = k_ref[0:KT, h, :]
            s = lax.dot_general(
                qh, kh, (((1,), (1,)), ((), ())),
                preferred_element_type=_F32,
            ) * SCALE
            w = jnp.where(mask, jnp.exp(s), 0.0)
            cols.append(jnp.dot(w, v_ref[0:KT, h, :],
                                preferred_element_type=_F32))
            dens.append(jnp.sum(w, axis=1, keepdims=True))
        den = jnp.concatenate(dens, axis=1)
        pad = jnp.zeros((BLK, PW - D - HQ), _F32)
        px_ref[...] = jnp.concatenate(cols + [den, pad], axis=1).astype(_BF16)
        pltpu.make_async_remote_copy(
            src_ref=px_ref,
            dst_ref=cx_ref.at[1],
            send_sem=s1_sems.at[1],
            recv_sem=cx_sems.at[1],
            device_id=(7,), device_id_type=_MESH,
        ).start()

    @pl.when(my == 0)
    def _():
        for k in range(N_DEV - 1):
            pltpu.make_async_remote_copy(
                src_ref=p32_ref,
                dst_ref=c0_ref.at[k],
                send_sem=s1_sems.at[0],
                recv_sem=c0_sems.at[k],
                device_id=(0,), device_id_type=_MESH,
            ).wait_recv()
        acc = c0_ref[0].astype(_F32)
        for k in range(1, N_DEV - 1):
            acc = acc + c0_ref[k].astype(_F32)
        bsum_ref[pl.ds(0, NGLOB)] = bsum_ref[pl.ds(0, NGLOB)] + acc

    @pl.when(my != 0)
    def _():
        pltpu.make_async_remote_copy(
            src_ref=px_ref,
            dst_ref=cx_ref.at[0],
            send_sem=s1_sems.at[0],
            recv_sem=cx_sems.at[0],
            device_id=(0,), device_id_type=_MESH,
        ).wait_recv()
        bsum_ref[...] = cx_ref[0].astype(_F32)

    @pl.when(my == 7)
    def _():
        pltpu.make_async_remote_copy(
            src_ref=px_ref,
            dst_ref=cx_ref.at[1],
            send_sem=s1_sems.at[0],
            recv_sem=cx_sems.at[1],
            device_id=(0,), device_id_type=_MESH,
        ).wait_recv()
        bsum_ref[...] = bsum_ref[...] + cx_ref[1].astype(_F32)

    @pl.when(my == 0)
    def _():
        for t in range(1, N_DEV):
            pltpu.make_async_remote_copy(
                src_ref=pstage_ref.at[t - 1],
                dst_ref=cx_ref.at[0],
                send_sem=s0_sems.at[t - 1],
                recv_sem=cx_sems.at[0],
                device_id=(t,), device_id_type=_MESH,
            ).wait_send()

    @pl.when(my != 0)
    def _():
        pltpu.make_async_remote_copy(
            src_ref=p32_ref,
            dst_ref=c0_ref.at[0],
            send_sem=s1_sems.at[0],
            recv_sem=c0_sems.at[0],
            device_id=(0,), device_id_type=_MESH,
        ).wait_send()

    @pl.when(my == 1)
    def _():
        pltpu.make_async_remote_copy(
            src_ref=px_ref,
            dst_ref=cx_ref.at[1],
            send_sem=s1_sems.at[1],
            recv_sem=cx_sems.at[1],
            device_id=(7,), device_id_type=_MESH,
        ).wait_send()

    _combine_finalize(my, wo_ref, o_ref, bsum_ref, agsrc_ref, agdst_ref,
                      s1_sems, ag_send_sems, ag_recv_sems)

    @functools.partial(pl.run_scoped, sem2=pltpu.SemaphoreType.REGULAR)
    def _(sem2):
        for p in range(N_DEV):
            @pl.when(my != p)
            def _(p=p):
                pl.semaphore_signal(
                    sem2, inc=1, device_id=(p,), device_id_type=_MESH)
        pl.semaphore_wait(sem2, N_DEV - 1)


def kernel(x, Wq, K_ext, V_ext, Wo):
    vmem = pl.BlockSpec(memory_space=pltpu.VMEM)

    return pl.pallas_call(
        _fused_body,
        out_shape=jax.ShapeDtypeStruct((1, SQ, D), jnp.float32),
        in_specs=[vmem] * 5,
        out_specs=vmem,
        scratch_shapes=[
            pltpu.VMEM((N_DEV - 1, BLK, PW), _BF16),
            pltpu.VMEM((NGLOB, PW), _BF16),
            pltpu.VMEM((BLK, PW), _BF16),
            pltpu.VMEM((N_DEV - 1, NGLOB, PW), _BF16),
            pltpu.VMEM((2, BLK, PW), _BF16),
            pltpu.VMEM((BLK, PW), jnp.float32),
            pltpu.VMEM((BLK, D), _BF16),
            pltpu.VMEM((N_DEV, BLK, D), _BF16),
            pltpu.SemaphoreType.DMA((N_DEV - 1,)),
            pltpu.SemaphoreType.DMA((2,)),
            pltpu.SemaphoreType.DMA((N_DEV - 1,)),
            pltpu.SemaphoreType.DMA((2,)),
            pltpu.SemaphoreType.DMA((N_DEV,)),
            pltpu.SemaphoreType.DMA((N_DEV,)),
        ],
        compiler_params=pltpu.CompilerParams(collective_id=0),
    )(x, Wq, K_ext[0], V_ext[0], Wo)


# device time: 54144 ns/iter; 1.1412x vs baseline; 1.1412x over previous
import functools

import jax
import jax.numpy as jnp
from jax import lax
from jax.experimental import pallas as pl
from jax.experimental.pallas import tpu as pltpu

N_DEV = 8
SQ = 1024
SKV_SH = 1024
HQ = 8
DH = 128
D = HQ * DH
PW = D + 128
BLK = SQ // N_DEV
KT = 128
NKT = SKV_SH // KT
SCALE = 0.08838834764831843
BAND = 128
NGLOB = 32

_MESH = pl.DeviceIdType.MESH
_BF16 = jnp.bfloat16
_F32 = jnp.float32


def _block_partial_dev0(q, k_ref, v_ref, t):
    if t == 0:
        lo, n = 0, SKV_SH
    elif t == 1:
        lo, n = 0, 3 * KT
    else:
        lo = (t - 1) * KT
        n = min(t + 2, NKT) * KT - lo
    glob_extra = t >= 2
    qi = t * BLK + lax.broadcasted_iota(jnp.int32, (BLK, n), 0)
    kj = lo + lax.broadcasted_iota(jnp.int32, (BLK, n), 1)
    mask = (jnp.abs(qi - kj) <= BAND) | (kj < NGLOB) | (qi < NGLOB)
    cols = []
    dens = []
    for h in range(HQ):
        qh = q[t * BLK:(t + 1) * BLK, h * DH:(h + 1) * DH]
        kb = k_ref[lo:lo + n, h * DH:(h + 1) * DH]
        s = lax.dot_general(
            qh, kb, (((1,), (1,)), ((), ())),
            preferred_element_type=_F32,
        ) * SCALE
        w = jnp.where(mask, jnp.exp(s), 0.0)
        num_h = jnp.dot(w, v_ref[lo:lo + n, h * DH:(h + 1) * DH],
                        preferred_element_type=_F32)
        den_h = jnp.sum(w, axis=1, keepdims=True)
        if glob_extra:
            kg = k_ref[0:NGLOB, h * DH:(h + 1) * DH]
            sg = lax.dot_general(
                qh, kg, (((1,), (1,)), ((), ())),
                preferred_element_type=_F32,
            ) * SCALE
            wg = jnp.exp(sg)
            num_h = num_h + jnp.dot(wg, v_ref[0:NGLOB, h * DH:(h + 1) * DH],
                                    preferred_element_type=_F32)
            den_h = den_h + jnp.sum(wg, axis=1, keepdims=True)
        cols.append(num_h)
        dens.append(den_h)
    den = jnp.concatenate(dens, axis=1)
    pad = jnp.zeros((BLK, PW - D - HQ), _F32)
    return jnp.concatenate(cols + [den, pad], axis=1).astype(_BF16)


def _combine_finalize(my, wo_ref, o_ref, bsum_ref, agsrc_ref, agdst_ref,
                      s1_sems, ag_send_sems, ag_recv_sems):
    parts = []
    for h in range(HQ):
        n = bsum_ref[:, h * DH:(h + 1) * DH]
        d = bsum_ref[:, D + h:D + h + 1]
        parts.append(n / d)
    ctx = jnp.concatenate(parts, axis=1)
    oblk = jnp.dot(ctx, wo_ref[...], preferred_element_type=_F32)

    o_ref[0, pl.ds(my * BLK, BLK)] = oblk
    agsrc_ref[...] = oblk.astype(_BF16)

    for t in range(N_DEV):
        @pl.when(my != t)
        def _(t=t):
            pltpu.make_async_remote_copy(
                src_ref=agsrc_ref,
                dst_ref=agdst_ref.at[my],
                send_sem=ag_send_sems.at[t],
                recv_sem=ag_recv_sems.at[my],
                device_id=(t,), device_id_type=_MESH,
            ).start()

    for k in range(N_DEV):
        @pl.when(my != k)
        def _(k=k):
            pltpu.make_async_remote_copy(
                src_ref=agsrc_ref,
                dst_ref=agdst_ref.at[k],
                send_sem=s1_sems.at[0],
                recv_sem=ag_recv_sems.at[k],
                device_id=(0,), device_id_type=_MESH,
            ).wait_recv()
            o_ref[0, k * BLK:(k + 1) * BLK] = agdst_ref[k].astype(_F32)

    for t in range(N_DEV):
        @pl.when(my != t)
        def _(t=t):
            pltpu.make_async_remote_copy(
                src_ref=agsrc_ref,
                dst_ref=agdst_ref.at[0],
                send_sem=ag_send_sems.at[t],
                recv_sem=ag_recv_sems.at[0],
                device_id=(t,), device_id_type=_MESH,
            ).wait_send()


def _fused_body(x_ref, wq_ref, k_ref, v_ref, wo_ref, o_ref,
                pstage_ref, p32_ref, px_ref,
                c0_ref, cx_ref, bsum_ref, agsrc_ref, agdst_ref,
                c0_sems, cx_sems, s0_sems, s1_sems,
                ag_send_sems, ag_recv_sems):
    my = lax.axis_index("i")

    barrier_sem = pltpu.get_barrier_semaphore()
    for p in range(N_DEV):
        @pl.when(my != p)
        def _(p=p):
            pl.semaphore_signal(
                barrier_sem, inc=1, device_id=(p,), device_id_type=_MESH)
    pl.semaphore_wait(barrier_sem, N_DEV - 1)

    @pl.when(my == 0)
    def _():
        q = jnp.dot(x_ref[0], wq_ref[...], preferred_element_type=_F32)
        for t in range(1, N_DEV):
            pstage_ref[t - 1] = _block_partial_dev0(q, k_ref, v_ref, t)
            pltpu.make_async_remote_copy(
                src_ref=pstage_ref.at[t - 1],
                dst_ref=cx_ref.at[0],
                send_sem=s0_sems.at[t - 1],
                recv_sem=cx_sems.at[0],
                device_id=(t,), device_id_type=_MESH,
            ).start()
        bsum_ref[...] = _block_partial_dev0(q, k_ref, v_ref, 0).astype(_F32)

    @pl.when(my != 0)
    def _():
        qg = jnp.dot(x_ref[0, pl.ds(0, NGLOB)], wq_ref[...],
                     preferred_element_type=_F32)
        cols = []
        dens = []
        for h in range(HQ):
            qh = qg[:, h * DH:(h + 1) * DH]
            kh = k_ref[:, h * DH:(h + 1) * DH]
            s = lax.dot_general(
                qh, kh, (((1,), (1,)), ((), ())),
                preferred_element_type=_F32,
            ) * SCALE
            w = jnp.exp(s)
            cols.append(jnp.dot(w, v_ref[:, h * DH:(h + 1) * DH],
                                preferred_element_type=_F32))
            dens.append(jnp.sum(w, axis=1, keepdims=True))
        den = jnp.concatenate(dens, axis=1)
        pad = jnp.zeros((NGLOB, PW - D - HQ), _F32)
        p32_ref[...] = jnp.concatenate(cols + [den, pad], axis=1).astype(_BF16)

    for p in range(1, N_DEV):
        @pl.when(my == p)
        def _(p=p):
            pltpu.make_async_remote_copy(
                src_ref=p32_ref,
                dst_ref=c0_ref.at[p - 1],
                send_sem=s1_sems.at[0],
                recv_sem=c0_sems.at[p - 1],
                device_id=(0,), device_id_type=_MESH,
            ).start()

    @pl.when(my == 1)
    def _():
        q9 = jnp.dot(x_ref[0, pl.ds(7 * BLK, BLK)], wq_ref[...],
                     preferred_element_type=_F32)
        qi = 7 * BLK + lax.broadcasted_iota(jnp.int32, (BLK, KT), 0)
        kj = SKV_SH + lax.broadcasted_iota(jnp.int32, (BLK, KT), 1)
        mask = jnp.abs(qi - kj) <= BAND
        cols = []
        dens = []
        for h in range(HQ):
            qh = q9[:, h * DH:(h + 1) * DH]
            kh = k_ref[0:KT, h * DH:(h + 1) * DH]
            s = lax.dot_general(
                qh, kh, (((1,), (1,)), ((), ())),
                preferred_element_type=_F32,
            ) * SCALE
            w = jnp.where(mask, jnp.exp(s), 0.0)
            cols.append(jnp.dot(w, v_ref[0:KT, h * DH:(h + 1) * DH],
                                preferred_element_type=_F32))
            dens.append(jnp.sum(w, axis=1, keepdims=True))
        den = jnp.concatenate(dens, axis=1)
        pad = jnp.zeros((BLK, PW - D - HQ), _F32)
        px_ref[...] = jnp.concatenate(cols + [den, pad], axis=1).astype(_BF16)
        pltpu.make_async_remote_copy(
            src_ref=px_ref,
            dst_ref=cx_ref.at[1],
            send_sem=s1_sems.at[1],
            recv_sem=cx_sems.at[1],
            device_id=(7,), device_id_type=_MESH,
        ).start()

    @pl.when(my == 0)
    def _():
        for k in range(N_DEV - 1):
            pltpu.make_async_remote_copy(
                src_ref=p32_ref,
                dst_ref=c0_ref.at[k],
                send_sem=s1_sems.at[0],
                recv_sem=c0_sems.at[k],
                device_id=(0,), device_id_type=_MESH,
            ).wait_recv()
        acc = c0_ref[0].astype(_F32)
        for k in range(1, N_DEV - 1):
            acc = acc + c0_ref[k].astype(_F32)
        bsum_ref[pl.ds(0, NGLOB)] = bsum_ref[pl.ds(0, NGLOB)] + acc

    @pl.when(my != 0)
    def _():
        pltpu.make_async_remote_copy(
            src_ref=px_ref,
            dst_ref=cx_ref.at[0],
            send_sem=s1_sems.at[0],
            recv_sem=cx_sems.at[0],
            device_id=(0,), device_id_type=_MESH,
        ).wait_recv()
        bsum_ref[...] = cx_ref[0].astype(_F32)

    @pl.when(my == 7)
    def _():
        pltpu.make_async_remote_copy(
            src_ref=px_ref,
            dst_ref=cx_ref.at[1],
            send_sem=s1_sems.at[0],
            recv_sem=cx_sems.at[1],
            device_id=(0,), device_id_type=_MESH,
        ).wait_recv()
        bsum_ref[...] = bsum_ref[...] + cx_ref[1].astype(_F32)

    @pl.when(my == 0)
    def _():
        for t in range(1, N_DEV):
            pltpu.make_async_remote_copy(
                src_ref=pstage_ref.at[t - 1],
                dst_ref=cx_ref.at[0],
                send_sem=s0_sems.at[t - 1],
                recv_sem=cx_sems.at[0],
                device_id=(t,), device_id_type=_MESH,
            ).wait_send()

    @pl.when(my != 0)
    def _():
        pltpu.make_async_remote_copy(
            src_ref=p32_ref,
            dst_ref=c0_ref.at[0],
            send_sem=s1_sems.at[0],
            recv_sem=c0_sems.at[0],
            device_id=(0,), device_id_type=_MESH,
        ).wait_send()

    @pl.when(my == 1)
    def _():
        pltpu.make_async_remote_copy(
            src_ref=px_ref,
            dst_ref=cx_ref.at[1],
            send_sem=s1_sems.at[1],
            recv_sem=cx_sems.at[1],
            device_id=(7,), device_id_type=_MESH,
        ).wait_send()

    _combine_finalize(my, wo_ref, o_ref, bsum_ref, agsrc_ref, agdst_ref,
                      s1_sems, ag_send_sems, ag_recv_sems)

    @functools.partial(pl.run_scoped, sem2=pltpu.SemaphoreType.REGULAR)
    def _(sem2):
        for p in range(N_DEV):
            @pl.when(my != p)
            def _(p=p):
                pl.semaphore_signal(
                    sem2, inc=1, device_id=(p,), device_id_type=_MESH)
        pl.semaphore_wait(sem2, N_DEV - 1)


def kernel(x, Wq, K_ext, V_ext, Wo):
    k2 = K_ext[0].reshape(SKV_SH, D)
    v2 = V_ext[0].reshape(SKV_SH, D)
    vmem = pl.BlockSpec(memory_space=pltpu.VMEM)

    return pl.pallas_call(
        _fused_body,
        out_shape=jax.ShapeDtypeStruct((1, SQ, D), jnp.float32),
        in_specs=[vmem] * 5,
        out_specs=vmem,
        scratch_shapes=[
            pltpu.VMEM((N_DEV - 1, BLK, PW), _BF16),
            pltpu.VMEM((NGLOB, PW), _BF16),
            pltpu.VMEM((BLK, PW), _BF16),
            pltpu.VMEM((N_DEV - 1, NGLOB, PW), _BF16),
            pltpu.VMEM((2, BLK, PW), _BF16),
            pltpu.VMEM((BLK, PW), jnp.float32),
            pltpu.VMEM((BLK, D), _BF16),
            pltpu.VMEM((N_DEV, BLK, D), _BF16),
            pltpu.SemaphoreType.DMA((N_DEV - 1,)),
            pltpu.SemaphoreType.DMA((2,)),
            pltpu.SemaphoreType.DMA((N_DEV - 1,)),
            pltpu.SemaphoreType.DMA((2,)),
            pltpu.SemaphoreType.DMA((N_DEV,)),
            pltpu.SemaphoreType.DMA((N_DEV,)),
        ],
        compiler_params=pltpu.CompilerParams(collective_id=0),
    )(x, Wq, k2, v2, Wo)
